# baseline (speedup 1.0000x reference)
import functools

import jax
import jax.numpy as jnp
from jax import lax
from jax.experimental import pallas as pl
from jax.experimental.pallas import tpu as pltpu
from jax.experimental.pallas import tpu_sc as plsc

C = 9
GAMMA = 2.0
N_BATCH = 8
H = 512
W = 512
BH = 512
SUB = 32
N_PIX = N_BATCH * H * W

NC = 2
NS = 16
L = 16
NW = NC * NS
CHUNK = N_PIX // NW
UNROLL = 8


def _focal_block_kernel(x_ref, t_ref, out_ref, s_ref, se_ref, xt_ref, *, nsteps):
    step = pl.program_id(0) * (H // BH) + pl.program_id(1)

    @pl.when(step == 0)
    def _init():
        s_ref[...] = jnp.zeros_like(s_ref)

    def strip(i, carry):
        r = i * SUB
        ts = t_ref[0, pl.ds(r, SUB), :]
        se = jnp.zeros((SUB, W), jnp.float32)
        xt = jnp.zeros((SUB, W), jnp.float32)
        for c in range(C):
            xc = x_ref[0, c, pl.ds(r, SUB), :]
            se = se + jnp.exp(xc)
            xt = jnp.where(ts == c, xc, xt)
        se_ref[pl.ds(r, SUB), :] = se
        xt_ref[pl.ds(r, SUB), :] = xt
        return carry

    lax.fori_loop(0, BH // SUB, strip, 0)

    t = t_ref[0]
    logp = xt_ref[...] - jnp.log(se_ref[...])
    p = jnp.exp(logp)
    omp = 1.0 - p
    contrib = -(omp * omp) * logp

    for c in range(C - 1):
        s_ref[c, :] += jnp.sum(jnp.where(t == c, contrib, 0.0), axis=0)
    s_ref[C - 1, :] += jnp.sum(contrib, axis=0)

    @pl.when(step == nsteps - 1)
    def _fin():
        s_sums = jnp.sum(s_ref[...], axis=1)
        s8 = s_sums[C - 1] - jnp.sum(s_sums[: C - 1])
        out_ref[0, :] = jnp.concatenate([s_sums[: C - 1], jnp.reshape(s8, (1,))])


@functools.partial(
    pl.kernel,
    mesh=plsc.VectorSubcoreMesh(core_axis_name="c", subcore_axis_name="s"),
    out_type=jax.ShapeDtypeStruct((NW, L), jnp.int32),
    scratch_types=[
        pltpu.VMEM((CHUNK,), jnp.int32),
        pltpu.VMEM((L * L,), jnp.int32),
        pltpu.VMEM((L,), jnp.int32),
    ],
    compiler_params=pltpu.CompilerParams(needs_layout_passes=False),
)
def _hist_sc_kernel(t_hbm, out_hbm, buf_v, table_v, acc_v):
    wid = lax.axis_index("s") * NC + lax.axis_index("c")
    base = wid * CHUNK
    pltpu.sync_copy(t_hbm.at[pl.ds(base, CHUNK)], buf_v)

    zero16 = jnp.zeros((L,), jnp.int32)
    for r in range(L):
        table_v[pl.ds(r * L, L)] = zero16

    lane_off = lax.iota(jnp.int32, 16) * L
    one16 = jnp.ones((L,), jnp.int32)

    def body(i, carry):
        for u in range(UNROLL):
            v = buf_v[pl.ds((i * UNROLL + u) * L, L)]
            plsc.addupdate_scatter(table_v, [v + lane_off], one16)
        return carry

    lax.fori_loop(0, CHUNK // (L * UNROLL), body, 0)

    acc = zero16
    for r in range(L):
        acc = acc + table_v[pl.ds(r * L, L)]
    acc_v[...] = acc
    pltpu.sync_copy(acc_v, out_hbm.at[wid])


@jax.jit
def kernel(inputs, targets):
    t32 = targets.astype(jnp.int32)
    count_rows = _hist_sc_kernel(t32.reshape(-1))

    nh = H // BH
    nsteps = N_BATCH * nh
    s = pl.pallas_call(
        functools.partial(_focal_block_kernel, nsteps=nsteps),
        grid=(N_BATCH, nh),
        in_specs=[
            pl.BlockSpec((1, C, BH, W), lambda b, h: (b, 0, h, 0)),
            pl.BlockSpec((1, BH, W), lambda b, h: (b, h, 0)),
        ],
        out_specs=pl.BlockSpec((1, C), lambda b, h: (0, 0)),
        out_shape=jax.ShapeDtypeStruct((1, C), jnp.float32),
        scratch_shapes=[
            pltpu.VMEM((C, W), jnp.float32),
            pltpu.VMEM((BH, W), jnp.float32),
            pltpu.VMEM((BH, W), jnp.float32),
        ],
        compiler_params=pltpu.CompilerParams(
            dimension_semantics=("arbitrary", "arbitrary"),
        ),
    )(inputs, t32)[0]

    cnt = jnp.sum(count_rows, axis=0)[:C].astype(jnp.float32)
    class_weights = 1.0 / jnp.log(1.1 + cnt / N_PIX)
    return jnp.dot(class_weights, s) / N_PIX


# speedup vs baseline: 1.8545x; 1.8545x over previous
import functools

import jax
import jax.numpy as jnp
from jax import lax
from jax.experimental import pallas as pl
from jax.experimental.pallas import tpu as pltpu

C = 9
GAMMA = 2.0
N_BATCH = 8
H = 512
W = 512
BH = 512
SUB = 32
N_PIX = N_BATCH * H * W


def _focal_block_kernel(
    x_ref, t_ref, out_ref, s_ref, n_ref, acc_ref, se_ref, xt_ref, *, nsteps
):
    step = pl.program_id(0) * (H // BH) + pl.program_id(1)

    @pl.when(step == 0)
    def _init():
        s_ref[...] = jnp.zeros_like(s_ref)
        n_ref[...] = jnp.zeros_like(n_ref)
        acc_ref[...] = jnp.zeros_like(acc_ref)

    def strip(i, carry):
        r = i * SUB
        ts = t_ref[0, pl.ds(r, SUB), :]
        se = jnp.zeros((SUB, W), jnp.float32)
        xt = jnp.zeros((SUB, W), jnp.float32)
        for c in range(C):
            xc = x_ref[0, c, pl.ds(r, SUB), :]
            se = se + jnp.exp(xc)
            xt = jnp.where(ts == c, xc, xt)
        se_ref[pl.ds(r, SUB), :] = se
        xt_ref[pl.ds(r, SUB), :] = xt
        return carry

    lax.fori_loop(0, BH // SUB, strip, 0)

    t = t_ref[0]
    logp = xt_ref[...] - jnp.log(se_ref[...])
    p = jnp.exp(logp)
    omp = 1.0 - p
    contrib = -(omp * omp) * logp

    for c in range(C - 1):
        s_ref[c, :] += jnp.sum(jnp.where(t == c, contrib, 0.0), axis=0)
    s_ref[C - 1, :] += jnp.sum(contrib, axis=0)

    acc_ref[...] += jnp.left_shift(jnp.int32(1), 3 * t)

    @pl.when((step % 7 == 6) | (step == nsteps - 1))
    def _flush():
        a = acc_ref[...]
        for c in range(C - 1):
            n_ref[c, :] += jnp.sum((a >> (3 * c)) & 7, axis=0)
        acc_ref[...] = jnp.zeros_like(acc_ref)

    @pl.when(step == nsteps - 1)
    def _fin():
        s_sums = jnp.sum(s_ref[...], axis=1)
        s8 = s_sums[C - 1] - jnp.sum(s_sums[: C - 1])
        out_ref[0, :] = jnp.concatenate([s_sums[: C - 1], jnp.reshape(s8, (1,))])
        n_sums = jnp.sum(n_ref[...], axis=1).astype(jnp.float32)
        n8 = jnp.float32(N_PIX) - jnp.sum(n_sums[: C - 1])
        out_ref[1, :] = jnp.concatenate([n_sums[: C - 1], jnp.reshape(n8, (1,))])


@jax.jit
def kernel(inputs, targets):
    nh = H // BH
    nsteps = N_BATCH * nh
    partials = pl.pallas_call(
        functools.partial(_focal_block_kernel, nsteps=nsteps),
        grid=(N_BATCH, nh),
        in_specs=[
            pl.BlockSpec((1, C, BH, W), lambda b, h: (b, 0, h, 0)),
            pl.BlockSpec((1, BH, W), lambda b, h: (b, h, 0)),
        ],
        out_specs=pl.BlockSpec((2, C), lambda b, h: (0, 0)),
        out_shape=jax.ShapeDtypeStruct((2, C), jnp.float32),
        scratch_shapes=[
            pltpu.VMEM((C, W), jnp.float32),
            pltpu.VMEM((C, W), jnp.int32),
            pltpu.VMEM((BH, W), jnp.int32),
            pltpu.VMEM((BH, W), jnp.float32),
            pltpu.VMEM((BH, W), jnp.float32),
        ],
        compiler_params=pltpu.CompilerParams(
            dimension_semantics=("arbitrary", "arbitrary"),
        ),
    )(inputs, targets.astype(jnp.int32))
    s = partials[0]
    cnt = partials[1]
    class_weights = 1.0 / jnp.log(1.1 + cnt / N_PIX)
    return jnp.dot(class_weights, s) / N_PIX
